# baseline (speedup 1.0000x reference)
import functools

import jax
import jax.numpy as jnp
from jax import lax
from jax.experimental import pallas as pl
from jax.experimental.pallas import tpu as pltpu
from jax.experimental.pallas import tpu_sc as plsc

B = 4
H = W = 192
C = 96
HW = H * W
P = (2 * H) * (2 * W)
CB = 2
NBLK = C // CB
NC = 2
NS = 16
LANES = 16
NQ = NS // CB
POSQ = HW // NQ
NELEM = POSQ
SLAB = CB * P
SLICE = SLAB // NS
TPC = NBLK // NC

TBLK_IN = 2048
TBLK_OUT = 4096


def _sc_body(mask_hbm, upd_hbm, out_hbm,
             mbuf0, mbuf1, ubuf0, ubuf1, midx0, midx1, zbuf, accA, accB,
             msem, usem, ssem, dsemA, dsemB, zsemA, zsemB):
  cid = lax.axis_index("c")
  sid = lax.axis_index("s")
  kch = lax.bitwise_and(sid, CB - 1)
  q = lax.shift_right_logical(sid, 1)
  kd = lax.shift_right_logical(sid, 3)
  rd = lax.bitwise_and(sid, NQ - 1) * SLICE

  zeros16 = jnp.zeros((LANES,), jnp.float32)

  def zfill(i, _):
    zbuf[pl.ds(i * LANES, LANES)] = zeros16
    return 0
  lax.fori_loop(0, SLICE // LANES, zfill, 0)

  def zero_start(acc, zsem):
    pltpu.async_copy(zbuf, acc.at[pl.ds(sid * SLICE, SLICE)], zsem)

  def zero_wait(acc, zsem):
    pltpu.make_async_copy(
        zbuf, acc.at[pl.ds(sid * SLICE, SLICE)], zsem
    ).wait()

  def src_slices(t):
    c = (t * NC + cid) * CB + kch
    pos0 = q * POSQ
    return c, pos0

  def start_in(t, mb, ub):
    c, pos0 = src_slices(t)
    pltpu.async_copy(mask_hbm.at[c, pl.ds(pos0, POSQ)], mb, msem)
    pltpu.async_copy(upd_hbm.at[c, pl.ds(pos0, POSQ)], ub, usem)

  def wait_in(t, mb, ub):
    c, pos0 = src_slices(t)
    pltpu.make_async_copy(
        mask_hbm.at[c, pl.ds(pos0, POSQ)], mb, msem
    ).wait()
    pltpu.make_async_copy(
        upd_hbm.at[c, pl.ds(pos0, POSQ)], ub, usem
    ).wait()

  cvec = jnp.full((LANES,), C, jnp.int32)

  def compute_idx(mb, mx):
    for k in range(CB):
      @pl.when(kch == k)
      def _(k=k):
        kvec = jnp.full((LANES,), k * P, jnp.int32)

        def compute(i, _):
          m = mb[pl.ds(i * LANES, LANES)]
          mx[pl.ds(i * LANES, LANES)] = lax.div(m, cvec) + kvec
          return 0
        lax.fori_loop(0, NELEM // LANES, compute, 0)

  def drain_ref(t):
    c0 = (t * NC + cid) * CB
    return out_hbm.at[c0 + kd, pl.ds(rd, SLICE)]

  def drain_start(t, acc, dsem):
    pltpu.async_copy(acc.at[pl.ds(sid * SLICE, SLICE)], drain_ref(t), dsem)

  def drain_wait(t, acc, dsem):
    pltpu.make_async_copy(
        acc.at[pl.ds(sid * SLICE, SLICE)], drain_ref(t), dsem
    ).wait()

  zero_start(accA, zsemA)
  zero_start(accB, zsemB)
  start_in(0, mbuf0, ubuf0)
  wait_in(0, mbuf0, ubuf0)
  compute_idx(mbuf0, midx0)
  start_in(1, mbuf1, ubuf1)

  def step(t, cur, nxt):
    mb_c, ub_c, mx_c, acc_c, dsem_c, zsem_c = cur
    mb_n, ub_n, mx_n, acc_n, dsem_n, zsem_n = nxt

    zero_wait(acc_c, zsem_c)
    plsc.subcore_barrier()

    pltpu.async_copy(ub_c, acc_c.at[mx_c], ssem, add=True)
    tn = lax.min(t + 1, TPC - 1)
    wait_in(tn, mb_n, ub_n)
    compute_idx(mb_n, mx_n)
    pltpu.make_async_copy(ub_c, acc_c.at[mx_c], ssem).wait()
    plsc.subcore_barrier()

    drain_start(t, acc_c, dsem_c)

    @pl.when(t > 0)
    def _():
      drain_wait(t - 1, acc_n, dsem_n)
      zero_start(acc_n, zsem_n)

    start_in(lax.min(t + 2, TPC - 1), mb_c, ub_c)
    return 0

  bufs0 = (mbuf0, ubuf0, midx0, accA, dsemA, zsemA)
  bufs1 = (mbuf1, ubuf1, midx1, accB, dsemB, zsemB)

  def task(t, _):
    even = lax.bitwise_and(t, 1) == 0

    @pl.when(even)
    def _():
      step(t, bufs0, bufs1)

    @pl.when(~even)
    def _():
      step(t, bufs1, bufs0)

    return 0

  lax.fori_loop(0, TPC, task, 0)

  drain_wait(TPC - 1, accB, dsemB)
  zero_wait(accA, zsemA)
  wait_in(TPC - 1, mbuf1, ubuf1)


def _sc_scatter(mask_cm, upd_cm):
  mesh = plsc.VectorSubcoreMesh(
      core_axis_name="c", subcore_axis_name="s", num_cores=NC, num_subcores=NS
  )
  return pl.kernel(
      _sc_body,
      out_type=jax.ShapeDtypeStruct((C, P), jnp.float32),
      mesh=mesh,
      scratch_types=[
          pltpu.VMEM((NELEM,), jnp.int32),
          pltpu.VMEM((NELEM,), jnp.int32),
          pltpu.VMEM((NELEM,), jnp.float32),
          pltpu.VMEM((NELEM,), jnp.float32),
          pltpu.VMEM((NELEM,), jnp.int32),
          pltpu.VMEM((NELEM,), jnp.int32),
          pltpu.VMEM((SLICE,), jnp.float32),
          pltpu.VMEM_SHARED((SLAB,), jnp.float32),
          pltpu.VMEM_SHARED((SLAB,), jnp.float32),
          pltpu.SemaphoreType.DMA,
          pltpu.SemaphoreType.DMA,
          pltpu.SemaphoreType.DMA,
          pltpu.SemaphoreType.DMA,
          pltpu.SemaphoreType.DMA,
          pltpu.SemaphoreType.DMA,
          pltpu.SemaphoreType.DMA,
      ],
  )(mask_cm, upd_cm)


def _tin_body(m_ref, u_ref, mt_ref, ut_ref):
  mt_ref[...] = jnp.swapaxes(m_ref[...], 0, 1)
  ut_ref[...] = jnp.swapaxes(u_ref[...], 0, 1)


def _transpose_in(mask_b, upd_b):
  grid = (HW // TBLK_IN,)
  return pl.pallas_call(
      _tin_body,
      grid=grid,
      in_specs=[
          pl.BlockSpec((TBLK_IN, C), lambda i: (i, 0)),
          pl.BlockSpec((TBLK_IN, C), lambda i: (i, 0)),
      ],
      out_specs=[
          pl.BlockSpec((C, TBLK_IN), lambda i: (0, i)),
          pl.BlockSpec((C, TBLK_IN), lambda i: (0, i)),
      ],
      out_shape=[
          jax.ShapeDtypeStruct((C, HW), jnp.int32),
          jax.ShapeDtypeStruct((C, HW), jnp.float32),
      ],
  )(mask_b, upd_b)


def _tout_body(o_ref, ot_ref):
  ot_ref[...] = jnp.swapaxes(o_ref[...], 0, 1)


def _transpose_out(out_cm):
  grid = (P // TBLK_OUT,)
  return pl.pallas_call(
      _tout_body,
      grid=grid,
      in_specs=[pl.BlockSpec((C, TBLK_OUT), lambda i: (0, i))],
      out_specs=pl.BlockSpec((TBLK_OUT, C), lambda i: (i, 0)),
      out_shape=jax.ShapeDtypeStruct((P, C), jnp.float32),
  )(out_cm)


@jax.jit
def kernel(updates, mask):
  mask3 = mask.astype(jnp.int32).reshape(B, HW, C)
  upd3 = updates.reshape(B, HW, C)
  outs = []
  for b in range(B):
    mask_cm, upd_cm = _transpose_in(mask3[b], upd3[b])
    out_cm = _sc_scatter(mask_cm, upd_cm)
    outs.append(_transpose_out(out_cm))
  return jnp.stack(outs).reshape(B, 2 * H, 2 * W, C)


# speedup vs baseline: 1.2318x; 1.2318x over previous
import jax
import jax.numpy as jnp
from jax import lax
from jax.experimental import pallas as pl
from jax.experimental.pallas import tpu as pltpu
from jax.experimental.pallas import tpu_sc as plsc

B = 4
H = W = 192
C = 96
HW = H * W
P = (2 * H) * (2 * W)
CB = 2
NBLK = C // CB
NC = 2
NS = 16
LANES = 16
NQ = NS // CB
POSQ = HW // NQ
NELEM = POSQ
SLAB = CB * P
SLICE = SLAB // NS
NTASK = B * NBLK
TPC = NTASK // NC


def _body(mask_hbm, upd_hbm, out_hbm,
          mbuf0, mbuf1, ubuf0, ubuf1, midx0, midx1, zbuf, accA, accB,
          msem, usem, ssem, dsemA, dsemB, zsemA, zsemB):
  cid = lax.axis_index("c")
  sid = lax.axis_index("s")
  kch = lax.bitwise_and(sid, CB - 1)
  q = lax.shift_right_logical(sid, 1)
  kd = lax.shift_right_logical(sid, 3)
  rd = lax.bitwise_and(sid, NQ - 1) * SLICE

  zeros16 = jnp.zeros((LANES,), jnp.float32)

  def zfill(i, _):
    zbuf[pl.ds(i * LANES, LANES)] = zeros16
    return 0
  lax.fori_loop(0, SLICE // LANES, zfill, 0)

  def zero_start(acc, zsem):
    pltpu.async_copy(zbuf, acc.at[pl.ds(sid * SLICE, SLICE)], zsem)

  def zero_wait(acc, zsem):
    pltpu.make_async_copy(
        zbuf, acc.at[pl.ds(sid * SLICE, SLICE)], zsem
    ).wait()

  def src_slices(t):
    task_id = t * NC + cid
    b = task_id // NBLK
    c = (task_id % NBLK) * CB + kch
    pos0 = q * POSQ
    return b, c, pos0

  def start_in(t, mb, ub):
    b, c, pos0 = src_slices(t)
    pltpu.async_copy(mask_hbm.at[b, c, pl.ds(pos0, POSQ)], mb, msem)
    pltpu.async_copy(upd_hbm.at[b, c, pl.ds(pos0, POSQ)], ub, usem)

  def wait_in(t, mb, ub):
    b, c, pos0 = src_slices(t)
    pltpu.make_async_copy(
        mask_hbm.at[b, c, pl.ds(pos0, POSQ)], mb, msem
    ).wait()
    pltpu.make_async_copy(
        upd_hbm.at[b, c, pl.ds(pos0, POSQ)], ub, usem
    ).wait()

  cvec = jnp.full((LANES,), C, jnp.int32)

  def compute_idx(mb, mx):
    for k in range(CB):
      @pl.when(kch == k)
      def _(k=k):
        kvec = jnp.full((LANES,), k * P, jnp.int32)

        def compute(i, _):
          m = mb[pl.ds(i * LANES, LANES)]
          mx[pl.ds(i * LANES, LANES)] = lax.div(m, cvec) + kvec
          return 0
        lax.fori_loop(0, NELEM // LANES, compute, 0)

  def drain_ref(t):
    task_id = t * NC + cid
    b = task_id // NBLK
    c0 = (task_id % NBLK) * CB
    return out_hbm.at[b, c0 + kd, pl.ds(rd, SLICE)]

  def drain_start(t, acc, dsem):
    pltpu.async_copy(acc.at[pl.ds(sid * SLICE, SLICE)], drain_ref(t), dsem)

  def drain_wait(t, acc, dsem):
    pltpu.make_async_copy(
        acc.at[pl.ds(sid * SLICE, SLICE)], drain_ref(t), dsem
    ).wait()

  zero_start(accA, zsemA)
  zero_start(accB, zsemB)
  start_in(0, mbuf0, ubuf0)
  wait_in(0, mbuf0, ubuf0)
  compute_idx(mbuf0, midx0)
  start_in(1, mbuf1, ubuf1)

  def step(t, cur, nxt):
    mb_c, ub_c, mx_c, acc_c, dsem_c, zsem_c = cur
    mb_n, ub_n, mx_n, acc_n, dsem_n, zsem_n = nxt

    zero_wait(acc_c, zsem_c)
    plsc.subcore_barrier()

    pltpu.async_copy(ub_c, acc_c.at[mx_c], ssem, add=True)
    tn = lax.min(t + 1, TPC - 1)
    wait_in(tn, mb_n, ub_n)
    compute_idx(mb_n, mx_n)
    pltpu.make_async_copy(ub_c, acc_c.at[mx_c], ssem).wait()
    plsc.subcore_barrier()

    drain_start(t, acc_c, dsem_c)

    @pl.when(t > 0)
    def _():
      drain_wait(t - 1, acc_n, dsem_n)
      zero_start(acc_n, zsem_n)

    start_in(lax.min(t + 2, TPC - 1), mb_c, ub_c)
    return 0

  bufs0 = (mbuf0, ubuf0, midx0, accA, dsemA, zsemA)
  bufs1 = (mbuf1, ubuf1, midx1, accB, dsemB, zsemB)

  def task(t, _):
    even = lax.bitwise_and(t, 1) == 0

    @pl.when(even)
    def _():
      step(t, bufs0, bufs1)

    @pl.when(~even)
    def _():
      step(t, bufs1, bufs0)

    return 0

  lax.fori_loop(0, TPC, task, 0)

  drain_wait(TPC - 1, accB, dsemB)
  zero_wait(accA, zsemA)
  wait_in(TPC - 1, mbuf1, ubuf1)


@jax.jit
def kernel(updates, mask):
  mask_t = jnp.transpose(mask.astype(jnp.int32).reshape(B, HW, C), (0, 2, 1))
  upd_t = jnp.transpose(updates.reshape(B, HW, C), (0, 2, 1))
  mesh = plsc.VectorSubcoreMesh(
      core_axis_name="c", subcore_axis_name="s", num_cores=NC, num_subcores=NS
  )
  out_t = pl.kernel(
      _body,
      out_type=jax.ShapeDtypeStruct((B, C, P), jnp.float32),
      mesh=mesh,
      scratch_types=[
          pltpu.VMEM((NELEM,), jnp.int32),
          pltpu.VMEM((NELEM,), jnp.int32),
          pltpu.VMEM((NELEM,), jnp.float32),
          pltpu.VMEM((NELEM,), jnp.float32),
          pltpu.VMEM((NELEM,), jnp.int32),
          pltpu.VMEM((NELEM,), jnp.int32),
          pltpu.VMEM((SLICE,), jnp.float32),
          pltpu.VMEM_SHARED((SLAB,), jnp.float32),
          pltpu.VMEM_SHARED((SLAB,), jnp.float32),
          pltpu.SemaphoreType.DMA,
          pltpu.SemaphoreType.DMA,
          pltpu.SemaphoreType.DMA,
          pltpu.SemaphoreType.DMA,
          pltpu.SemaphoreType.DMA,
          pltpu.SemaphoreType.DMA,
          pltpu.SemaphoreType.DMA,
      ],
  )(mask_t, upd_t)
  return jnp.transpose(out_t, (0, 2, 1)).reshape(B, 2 * H, 2 * W, C)
